# baseline (speedup 1.0000x reference)
import functools

import jax
import jax.numpy as jnp
from jax import lax
from jax.experimental import pallas as pl
from jax.experimental.pallas import tpu as pltpu

_EPS = 1e-5
_NB = 32


def _shift_lanes(x, s):
    return pltpu.roll(x, (-s) % x.shape[1], 1)


def _gen_masks(rows, lanes, width, length, dtype):
    lane = lax.broadcasted_iota(jnp.int32, (rows, lanes), 1)
    wpos = lane % width
    hpos = lane % length
    keep_l = (wpos != 0).astype(dtype)
    keep_r = (wpos != (width - 1)).astype(dtype)
    keep_t = (hpos >= width).astype(dtype)
    keep_b = (hpos < (length - width)).astype(dtype)
    return keep_l, keep_r, keep_t, keep_b


def _conv3x3(xcat, wall, m_scr, width, co):
    xm = _shift_lanes(xcat, -1) * m_scr[0]
    xp = _shift_lanes(xcat, 1) * m_scr[1]
    x3 = jnp.concatenate([xm, xcat, xp], axis=0)
    y3 = jnp.dot(wall, x3, preferred_element_type=jnp.float32)
    t0 = _shift_lanes(y3[0:co], -width) * m_scr[2].astype(jnp.float32)
    t2 = _shift_lanes(y3[2 * co:3 * co], width) * m_scr[3].astype(jnp.float32)
    return y3[co:2 * co] + t0 + t2


def _fused_kernel(nb, width, cnt, x_ref, w1_ref, gb_ref, w2_ref,
                  b2_ref, o_ref, y1_scr, st_scr, m_scr):
    ci, length = x_ref.shape[1], x_ref.shape[2]
    co = w1_ref.shape[0] // 3
    p = pl.program_id(0)
    i = pl.program_id(1)

    @pl.when((p == 0) & (i == 0))
    def _init_masks():
        kl, kr, kt, kb = _gen_masks(ci, nb * length, width, length,
                                    jnp.bfloat16)
        m_scr[0] = kl
        m_scr[1] = kr
        m_scr[2] = kt
        m_scr[3] = kb

    @pl.when(p == 0)
    def _phase0():
        xcat = jnp.concatenate(
            [x_ref[j].astype(jnp.bfloat16) for j in range(nb)], axis=1)
        y = _conv3x3(xcat, w1_ref[...], m_scr, width, co)
        yb = y.astype(jnp.bfloat16)
        y1_scr[i] = yb
        ssum = jnp.sum(y, axis=1, keepdims=True)
        ssq = jnp.sum(yb * yb, axis=1, keepdims=True, dtype=jnp.float32)
        st = jnp.concatenate([ssum, ssq], axis=1)
        prev = jnp.where(i == 0, 0.0, st_scr[...])
        st_scr[...] = prev + st

    @pl.when(p == 1)
    def _phase1():
        stats = st_scr[...]
        mean = stats[:, 0:1] / cnt
        var = jnp.maximum(stats[:, 1:2] / cnt - mean * mean, 0.0)
        scale = gb_ref[:, 0:1] * lax.rsqrt(var + _EPS)
        shift = gb_ref[:, 1:2] - mean * scale
        a = jnp.maximum(y1_scr[i] * scale + shift, 0.0)
        ab = a.astype(jnp.bfloat16)
        out = _conv3x3(ab, w2_ref[...], m_scr, width, co) + b2_ref[...]
        for j in range(nb):
            o_ref[j] = out[:, j * length:(j + 1) * length]


@jax.jit
def _forward(x_nchw, w1, b1, gamma, beta, w2, b2):
    n, ci, h, w = x_nchw.shape
    co = w1.shape[-1]
    length = h * w

    nb = _NB
    while n % nb:
        nb //= 2
    steps = n // nb

    x_r = x_nchw.reshape(n, ci, length).astype(jnp.float32)
    w1a = jnp.transpose(w1.astype(jnp.bfloat16), (0, 3, 1, 2)).reshape(
        3 * co, 3 * ci)
    w2a = jnp.transpose(w2.astype(jnp.bfloat16), (0, 3, 1, 2)).reshape(
        3 * co, 3 * co)
    b2c = b2.astype(jnp.float32).reshape(co, 1)
    gb = jnp.stack([gamma.astype(jnp.float32),
                    beta.astype(jnp.float32)], axis=1)

    cnt = float(n * h * w)
    body = functools.partial(_fused_kernel, nb, w, cnt)
    out = pl.pallas_call(
        body,
        out_shape=jax.ShapeDtypeStruct((n, co, length), jnp.float32),
        grid=(2, steps),
        in_specs=[
            pl.BlockSpec((nb, ci, length),
                         lambda p, i: ((1 - p) * i + p * (steps - 1), 0, 0)),
            pl.BlockSpec((3 * co, 3 * ci), lambda p, i: (0, 0)),
            pl.BlockSpec((co, 2), lambda p, i: (0, 0)),
            pl.BlockSpec((3 * co, 3 * co), lambda p, i: (0, 0)),
            pl.BlockSpec((co, 1), lambda p, i: (0, 0)),
        ],
        out_specs=pl.BlockSpec((nb, co, length), lambda p, i: (i * p, 0, 0)),
        scratch_shapes=[
            pltpu.VMEM((steps, co, nb * length), jnp.bfloat16),
            pltpu.VMEM((co, 2), jnp.float32),
            pltpu.VMEM((4, ci, nb * length), jnp.bfloat16),
        ],
        compiler_params=pltpu.CompilerParams(
            dimension_semantics=("arbitrary", "arbitrary")),
    )(x_r, w1a, gb, w2a, b2c)

    return out.reshape(n, co, h, w)


def kernel(x_nchw, w1, b1, gamma, beta, w2, b2):
    return _forward(x_nchw, w1, b1, gamma, beta, w2, b2)


# speedup vs baseline: 1.0096x; 1.0096x over previous
import functools

import jax
import jax.numpy as jnp
from jax import lax
from jax.experimental import pallas as pl
from jax.experimental.pallas import tpu as pltpu

_EPS = 1e-5
_NB = 32


def _shift_lanes(x, s):
    return pltpu.roll(x, (-s) % x.shape[1], 1)


def _gen_masks(rows, lanes, width, length, dtype):
    lane = lax.broadcasted_iota(jnp.int32, (rows, lanes), 1)
    wpos = lane % width
    hpos = lane % length
    keep_l = (wpos != 0).astype(dtype)
    keep_r = (wpos != (width - 1)).astype(dtype)
    keep_t = (hpos >= width).astype(dtype)
    keep_b = (hpos < (length - width)).astype(dtype)
    return keep_l, keep_r, keep_t, keep_b


def _conv3x3(xcat, wall, m_scr, width, co):
    xm = _shift_lanes(xcat, -1) * m_scr[0]
    xp = _shift_lanes(xcat, 1) * m_scr[1]
    x3 = jnp.concatenate([xm, xcat, xp], axis=0)
    y3 = jnp.dot(wall, x3, preferred_element_type=jnp.float32)
    t0 = _shift_lanes(y3[0:co], -width) * m_scr[2].astype(jnp.float32)
    t2 = _shift_lanes(y3[2 * co:3 * co], width) * m_scr[3].astype(jnp.float32)
    return y3[co:2 * co] + t0 + t2


def _fused_kernel(nb, width, cnt, x_ref, w1_ref, gb_ref, w2_ref,
                  b2_ref, o_ref, y1_scr, st_scr, m_scr):
    ci, length = x_ref.shape[1], x_ref.shape[2]
    co = w1_ref.shape[0] // 3
    p = pl.program_id(0)
    i = pl.program_id(1)

    @pl.when((p == 0) & (i == 0))
    def _init_masks():
        kl, kr, kt, kb = _gen_masks(ci, nb * length, width, length,
                                    jnp.bfloat16)
        m_scr[0] = kl
        m_scr[1] = kr
        m_scr[2] = kt
        m_scr[3] = kb

    @pl.when(p == 0)
    def _phase0():
        xcat = jnp.concatenate(
            [x_ref[j].astype(jnp.bfloat16) for j in range(nb)], axis=1)
        y = _conv3x3(xcat, w1_ref[...], m_scr, width, co)
        y1_scr[i] = y.astype(jnp.bfloat16)
        ssum = jnp.sum(y, axis=1, keepdims=True)
        ssq = jnp.sum(y * y, axis=1, keepdims=True)
        st = jnp.concatenate([ssum, ssq], axis=1)
        prev = jnp.where(i == 0, 0.0, st_scr[...])
        st_scr[...] = prev + st

    @pl.when(p == 1)
    def _phase1():
        stats = st_scr[...]
        mean = stats[:, 0:1] / cnt
        var = jnp.maximum(stats[:, 1:2] / cnt - mean * mean, 0.0)
        scale = gb_ref[:, 0:1] * lax.rsqrt(var + _EPS)
        shift = gb_ref[:, 1:2] - mean * scale
        a = jnp.maximum(y1_scr[i] * scale + shift, 0.0)
        ab = a.astype(jnp.bfloat16)
        out = _conv3x3(ab, w2_ref[...], m_scr, width, co) + b2_ref[...]
        for j in range(nb):
            o_ref[j] = out[:, j * length:(j + 1) * length]


@jax.jit
def _forward(x_nchw, w1, b1, gamma, beta, w2, b2):
    n, ci, h, w = x_nchw.shape
    co = w1.shape[-1]
    length = h * w

    nb = _NB
    while n % nb:
        nb //= 2
    steps = n // nb

    x_r = x_nchw.reshape(n, ci, length).astype(jnp.float32)
    w1a = jnp.transpose(w1.astype(jnp.bfloat16), (0, 3, 1, 2)).reshape(
        3 * co, 3 * ci)
    w2a = jnp.transpose(w2.astype(jnp.bfloat16), (0, 3, 1, 2)).reshape(
        3 * co, 3 * co)
    b2c = b2.astype(jnp.float32).reshape(co, 1)
    gb = jnp.stack([gamma.astype(jnp.float32),
                    beta.astype(jnp.float32)], axis=1)

    cnt = float(n * h * w)
    body = functools.partial(_fused_kernel, nb, w, cnt)
    out = pl.pallas_call(
        body,
        out_shape=jax.ShapeDtypeStruct((n, co, length), jnp.float32),
        grid=(2, steps),
        in_specs=[
            pl.BlockSpec((nb, ci, length),
                         lambda p, i: ((1 - p) * i + p * (steps - 1), 0, 0)),
            pl.BlockSpec((3 * co, 3 * ci), lambda p, i: (0, 0)),
            pl.BlockSpec((co, 2), lambda p, i: (0, 0)),
            pl.BlockSpec((3 * co, 3 * co), lambda p, i: (0, 0)),
            pl.BlockSpec((co, 1), lambda p, i: (0, 0)),
        ],
        out_specs=pl.BlockSpec((nb, co, length), lambda p, i: (i * p, 0, 0)),
        scratch_shapes=[
            pltpu.VMEM((steps, co, nb * length), jnp.bfloat16),
            pltpu.VMEM((co, 2), jnp.float32),
            pltpu.VMEM((4, ci, nb * length), jnp.bfloat16),
        ],
        compiler_params=pltpu.CompilerParams(
            dimension_semantics=("arbitrary", "arbitrary")),
    )(x_r, w1a, gb, w2a, b2c)

    return out.reshape(n, co, h, w)


def kernel(x_nchw, w1, b1, gamma, beta, w2, b2):
    return _forward(x_nchw, w1, b1, gamma, beta, w2, b2)
